# baseline (speedup 1.0000x reference)
import functools

import jax
import jax.numpy as jnp
from jax import lax
from jax.experimental import pallas as pl
from jax.experimental.pallas import tpu as pltpu
from jax.experimental.pallas import tpu_sc as plsc

N_NODES = 100000
N_ELEM = 128
ROWS = 160
CHUNK = ROWS * N_ELEM
N_CHUNKS = N_NODES // ROWS
NW = 32
N_ITERS = -(-N_CHUNKS // NW)
NBUF = 2


def _onehot_body(types_hbm, out_hbm, buf0, buf1, types_v, sem_t, sem0, sem1):
    wid = lax.axis_index("s") * 2 + lax.axis_index("c")
    bufs = (buf0, buf1)
    sems = (sem0, sem1)
    iota = lax.iota(jnp.int32, 16)
    ones = jnp.ones((16,), jnp.float32)
    zeros = jnp.zeros((16,), jnp.float32)

    for i in range(N_ITERS):
        c = wid + NW * i

        @pl.when(c < N_CHUNKS)
        def _():
            pltpu.make_async_copy(
                types_hbm.at[pl.ds(c * ROWS, ROWS)],
                types_v.at[pl.ds(i * ROWS, ROWS)],
                sem_t,
            ).start()

    for buf in bufs:
        def zbody(j, _, buf=buf):
            buf[pl.ds(j * 16, 16)] = zeros
            return 0
        lax.fori_loop(0, CHUNK // 16, zbody, 0)

    for i in range(N_ITERS):
        c = wid + NW * i

        @pl.when(c < N_CHUNKS)
        def _():
            pltpu.make_async_copy(
                types_hbm.at[pl.ds(0, ROWS)],
                types_v.at[pl.ds(i * ROWS, ROWS)],
                sem_t,
            ).wait()

    for i in range(N_ITERS):
        c = wid + NW * i
        b = i % NBUF

        @pl.when(c < N_CHUNKS)
        def _():
            if i >= NBUF:
                c_old = wid + NW * (i - NBUF)
                pltpu.make_async_copy(
                    bufs[b], out_hbm.at[pl.ds(c_old * CHUNK, CHUNK)], sems[b]
                ).wait()
                for j in range(ROWS // 16):
                    tv = types_v[pl.ds((i - NBUF) * ROWS + 16 * j, 16)]
                    idx = (16 * j + iota) * N_ELEM + tv
                    plsc.store_scatter(bufs[b], [idx], zeros)
            for j in range(ROWS // 16):
                tv = types_v[pl.ds(i * ROWS + 16 * j, 16)]
                idx = (16 * j + iota) * N_ELEM + tv
                plsc.store_scatter(bufs[b], [idx], ones)
            pltpu.make_async_copy(
                bufs[b], out_hbm.at[pl.ds(c * CHUNK, CHUNK)], sems[b]
            ).start()

    for b in range(NBUF):
        pltpu.make_async_copy(
            bufs[b], out_hbm.at[pl.ds(0, CHUNK)], sems[b]
        ).wait()


@jax.jit
def _onehot_sc(atomic_types):
    mesh = plsc.VectorSubcoreMesh(core_axis_name="c", subcore_axis_name="s")
    f = functools.partial(
        pl.kernel,
        mesh=mesh,
        compiler_params=pltpu.CompilerParams(
            needs_layout_passes=False,
            use_tc_tiling_on_sc=False,
        ),
        out_type=jax.ShapeDtypeStruct((N_NODES * N_ELEM,), jnp.float32),
        scratch_types=[
            pltpu.VMEM((CHUNK,), jnp.float32),
            pltpu.VMEM((CHUNK,), jnp.float32),
            pltpu.VMEM((N_ITERS * ROWS,), jnp.int32),
            pltpu.SemaphoreType.DMA,
            pltpu.SemaphoreType.DMA,
            pltpu.SemaphoreType.DMA,
        ],
    )(_onehot_body)
    return f(atomic_types)


def kernel(atomic_types, positions):
    del positions
    return _onehot_sc(atomic_types).reshape(N_NODES, N_ELEM)
